# baseline (speedup 1.0000x reference)
import functools

import numpy as np
import jax
import jax.numpy as jnp
from jax import lax
from jax.experimental import pallas as pl
from jax.experimental.pallas import tpu as pltpu
from jax.experimental.pallas import tpu_sc as plsc

NB = 32


def _sc_gather_build(V, D, B):
    info = plsc.get_sparse_core_info()
    NC, NS = info.num_cores, info.num_subcores
    NW = NC * NS
    assert D % info.num_lanes == 0 and B % (8 * NW) == 0
    b_per_w = B // NW
    mesh = plsc.VectorSubcoreMesh(core_axis_name="c", subcore_axis_name="s")

    @functools.partial(
        pl.kernel,
        mesh=mesh,
        compiler_params=pltpu.CompilerParams(use_tc_tiling_on_sc=False),
        out_type=jax.ShapeDtypeStruct((B, D), jnp.float32),
        scratch_types=[
            pltpu.VMEM((b_per_w,), jnp.int32),
            pltpu.VMEM((b_per_w, D), jnp.float32),
            pltpu.SemaphoreType.DMA,
        ],
    )
    def gather(table_hbm, idx_hbm, out_hbm, idx_v, rows_v, sem):
        wid = lax.axis_index("s") * NC + lax.axis_index("c")
        base = wid * b_per_w
        pltpu.sync_copy(idx_hbm.at[pl.ds(base, b_per_w)], idx_v)
        pltpu.async_copy(table_hbm.at[idx_v], rows_v, sem).wait()
        pltpu.sync_copy(rows_v, out_hbm.at[pl.ds(base, b_per_w)])

    return gather


def _tc_body(uid_ref, ue_ref, x_ref, wp_ref, bp_ref, bd_ref, bd2_ref, o_ref,
             o2_ref, wt_ref, wb_ref, br_ref, out_ref, fin_ref, e_scr,
             wtbd_scr, wpbbd_scr, bpbrbd_scr, *, S, R, WORD, E, RW,
             n_programs):
    i = pl.program_id(0)
    @pl.when(i == 0)
    def _():
        for s in range(E):
            t = jnp.dot(wb_ref[s], bd_ref[s],
                        preferred_element_type=jnp.float32)
            wtbd_scr[s] = jnp.dot(wt_ref[s], bd_ref[s],
                                  preferred_element_type=jnp.float32
                                  ).astype(jnp.bfloat16)
            wpbbd_scr[s] = jnp.dot(wp_ref[...], t,
                                   preferred_element_type=jnp.float32
                                   ).astype(jnp.bfloat16)
            bpbrbd_scr[s] = (
                jnp.dot(bp_ref[...], t, preferred_element_type=jnp.float32)
                + jnp.dot(br_ref[pl.ds(s, 1), :], bd_ref[s],
                          preferred_element_type=jnp.float32))
    uids = [uid_ref[i * NB + j] for j in range(NB)]
    sids = [lax.rem(u, E) for u in uids]
    for j in range(NB):
        sid = sids[j]
        ue_j = ue_ref[pl.ds(j, 1), :].astype(jnp.bfloat16)
        x = x_ref[pl.ds(j * S, S), :].astype(jnp.bfloat16)
        sc = (jnp.dot(x, wtbd_scr[sid], preferred_element_type=jnp.float32)
              + jnp.dot(ue_j, wpbbd_scr[sid],
                        preferred_element_type=jnp.float32)
              + bpbrbd_scr[sid])
        e = jnp.exp(sc)
        e_scr[j] = e.astype(jnp.bfloat16)
        if j == NB - 1:
            @pl.when(i == n_programs - 1)
            def _():
                d512 = jnp.dot(e, o2_ref[...],
                               preferred_element_type=jnp.float32)
                fin_ref[...] = e / d512
    for j in range(NB):
        eb = e_scr[j]
        un = jnp.dot(eb, bd2_ref[sids[j]],
                     preferred_element_type=jnp.float32)
        den = jnp.dot(eb, o_ref[...], preferred_element_type=jnp.float32)
        out_ref[pl.ds(j * S, S), :] = un / den


def kernel(inputs, user_id, user_emb_table, W_proc, b_proc, shard_memory,
           W_read, b_read):
    B, S, D_IN = inputs.shape
    V, EMB = user_emb_table.shape
    E, M, WORD = shard_memory.shape
    RW = W_read.shape[-1]
    R = RW // WORD
    D_PROC = W_proc.shape[-1]

    uid = user_id.astype(jnp.int32)

    ue = _sc_gather_build(V, EMB, B)(user_emb_table, uid)

    wt = W_read[:, :D_IN, :]
    wb = W_read[:, D_IN:, :]
    bp2 = b_proc.reshape(1, D_PROC)

    memT = jnp.swapaxes(shard_memory, 1, 2) * (1.0 / np.sqrt(WORD))
    bd = jnp.zeros((E, R, WORD, R, M), jnp.float32)
    bd2 = jnp.zeros((E, R, M, R, WORD), jnp.float32)
    for r in range(R):
        bd = bd.at[:, r, :, r, :].set(memT)
        bd2 = bd2.at[:, r, :, r, :].set(shard_memory)
    bd = bd.reshape(E, RW, R * M)
    bd2 = bd2.reshape(E, R * M, RW).astype(jnp.bfloat16)
    o_mat = jnp.kron(jnp.eye(R, dtype=jnp.bfloat16),
                     jnp.ones((M, WORD), jnp.bfloat16))
    o2_mat = jnp.kron(jnp.eye(R, dtype=jnp.float32),
                      jnp.ones((M, M), jnp.float32))

    n_programs = B // NB
    grid_spec = pltpu.PrefetchScalarGridSpec(
        num_scalar_prefetch=1,
        grid=(n_programs,),
        in_specs=[
            pl.BlockSpec((NB, EMB), lambda i, u: (i, 0)),
            pl.BlockSpec((NB * S, D_IN), lambda i, u: (i, 0)),
            pl.BlockSpec((EMB, D_PROC), lambda i, u: (0, 0)),
            pl.BlockSpec((1, D_PROC), lambda i, u: (0, 0)),
            pl.BlockSpec((E, RW, R * M), lambda i, u: (0, 0, 0)),
            pl.BlockSpec((E, R * M, RW), lambda i, u: (0, 0, 0)),
            pl.BlockSpec((R * M, RW), lambda i, u: (0, 0)),
            pl.BlockSpec((R * M, R * M), lambda i, u: (0, 0)),
            pl.BlockSpec((E, D_IN, RW), lambda i, u: (0, 0, 0)),
            pl.BlockSpec((E, D_PROC, RW), lambda i, u: (0, 0, 0)),
            pl.BlockSpec((E, RW), lambda i, u: (0, 0)),
        ],
        out_specs=[
            pl.BlockSpec((NB * S, RW), lambda i, u: (i, 0)),
            pl.BlockSpec((S, R * M), lambda i, u: (0, 0)),
        ],
        scratch_shapes=[
            pltpu.VMEM((NB, S, R * M), jnp.bfloat16),
            pltpu.VMEM((E, D_IN, R * M), jnp.bfloat16),
            pltpu.VMEM((E, EMB, R * M), jnp.bfloat16),
            pltpu.VMEM((E, 1, R * M), jnp.float32),
        ],
    )
    body = functools.partial(_tc_body, S=S, R=R, WORD=WORD, E=E, RW=RW,
                             n_programs=n_programs)
    read_flat, fin = pl.pallas_call(
        body,
        grid_spec=grid_spec,
        out_shape=[
            jax.ShapeDtypeStruct((B * S, RW), jnp.float32),
            jax.ShapeDtypeStruct((S, R * M), jnp.float32),
        ],
    )(uid, ue, inputs.reshape(B * S, D_IN), W_proc, bp2, bd, bd2, o_mat,
      o2_mat, wt, wb, b_read)

    read_words = read_flat.reshape(B, S, R, WORD)
    final_state = fin.reshape(S, R, M)
    return (read_words, final_state)


# speedup vs baseline: 1.0975x; 1.0975x over previous
import functools

import numpy as np
import jax
import jax.numpy as jnp
from jax import lax
from jax.experimental import pallas as pl
from jax.experimental.pallas import tpu as pltpu
from jax.experimental.pallas import tpu_sc as plsc

NB = 32


def _sc_gather_build(V, D, B):
    info = plsc.get_sparse_core_info()
    NC, NS = info.num_cores, info.num_subcores
    NW = NC * NS
    assert D % info.num_lanes == 0 and B % (8 * NW) == 0
    b_per_w = B // NW
    mesh = plsc.VectorSubcoreMesh(core_axis_name="c", subcore_axis_name="s")

    @functools.partial(
        pl.kernel,
        mesh=mesh,
        compiler_params=pltpu.CompilerParams(use_tc_tiling_on_sc=False),
        out_type=jax.ShapeDtypeStruct((B, D), jnp.float32),
        scratch_types=[
            pltpu.VMEM((b_per_w,), jnp.int32),
            pltpu.VMEM((b_per_w, D), jnp.float32),
            pltpu.SemaphoreType.DMA,
        ],
    )
    def gather(table_hbm, idx_hbm, out_hbm, idx_v, rows_v, sem):
        wid = lax.axis_index("s") * NC + lax.axis_index("c")
        base = wid * b_per_w
        pltpu.sync_copy(idx_hbm.at[pl.ds(base, b_per_w)], idx_v)
        pltpu.async_copy(table_hbm.at[idx_v], rows_v, sem).wait()
        pltpu.sync_copy(rows_v, out_hbm.at[pl.ds(base, b_per_w)])

    return gather


def _tc_body(uid_ref, ue_ref, x_ref, wp_ref, bp_ref, bd_ref, bd2_ref, o_ref,
             o2_ref, wt_ref, wb_ref, br_ref, out_ref, fin_ref, e_scr,
             wtbd_scr, wpbbd_scr, bpbrbd_scr, *, S, R, WORD, E, RW,
             n_programs):
    i = pl.program_id(0)
    @pl.when(i == 0)
    def _():
        for s in range(E):
            t = jnp.dot(wb_ref[s], bd_ref[s],
                        preferred_element_type=jnp.float32)
            wtbd_scr[s] = jnp.dot(wt_ref[s], bd_ref[s],
                                  preferred_element_type=jnp.float32
                                  ).astype(jnp.bfloat16)
            wpbbd_scr[s] = jnp.dot(wp_ref[...], t,
                                   preferred_element_type=jnp.float32
                                   ).astype(jnp.bfloat16)
            bpbrbd_scr[s] = (
                jnp.dot(bp_ref[...], t, preferred_element_type=jnp.float32)
                + jnp.dot(br_ref[pl.ds(s, 1), :], bd_ref[s],
                          preferred_element_type=jnp.float32))
    uids = [uid_ref[i * NB + j] for j in range(NB)]
    sids = [lax.rem(u, E) for u in uids]
    for j in range(NB):
        sid = sids[j]
        ue_j = ue_ref[pl.ds(j, 1), :].astype(jnp.bfloat16)
        x = x_ref[j].astype(jnp.bfloat16)
        sc = (jnp.dot(x, wtbd_scr[sid], preferred_element_type=jnp.float32)
              + jnp.dot(ue_j, wpbbd_scr[sid],
                        preferred_element_type=jnp.float32)
              + bpbrbd_scr[sid])
        e = jnp.exp(sc)
        e_scr[j] = e.astype(jnp.bfloat16)
        if j == NB - 1:
            @pl.when(i == n_programs - 1)
            def _():
                d512 = jnp.dot(e, o2_ref[...],
                               preferred_element_type=jnp.float32)
                fin_ref[...] = e / d512
    for j in range(NB):
        eb = e_scr[j]
        un = jnp.dot(eb, bd2_ref[sids[j]],
                     preferred_element_type=jnp.float32)
        den = jnp.dot(eb, o_ref[...], preferred_element_type=jnp.float32)
        out_ref[j] = (un / den).reshape(S, R, WORD)


def kernel(inputs, user_id, user_emb_table, W_proc, b_proc, shard_memory,
           W_read, b_read):
    B, S, D_IN = inputs.shape
    V, EMB = user_emb_table.shape
    E, M, WORD = shard_memory.shape
    RW = W_read.shape[-1]
    R = RW // WORD
    D_PROC = W_proc.shape[-1]

    uid = user_id.astype(jnp.int32)

    ue = _sc_gather_build(V, EMB, B)(user_emb_table, uid)

    wt = W_read[:, :D_IN, :]
    wb = W_read[:, D_IN:, :]
    bp2 = b_proc.reshape(1, D_PROC)

    memT = jnp.swapaxes(shard_memory, 1, 2) * (1.0 / np.sqrt(WORD))
    bd = jnp.zeros((E, R, WORD, R, M), jnp.float32)
    bd2 = jnp.zeros((E, R, M, R, WORD), jnp.float32)
    for r in range(R):
        bd = bd.at[:, r, :, r, :].set(memT)
        bd2 = bd2.at[:, r, :, r, :].set(shard_memory)
    bd = bd.reshape(E, RW, R * M)
    bd2 = bd2.reshape(E, R * M, RW).astype(jnp.bfloat16)
    o_mat = jnp.kron(jnp.eye(R, dtype=jnp.bfloat16),
                     jnp.ones((M, WORD), jnp.bfloat16))
    o2_mat = jnp.kron(jnp.eye(R, dtype=jnp.float32),
                      jnp.ones((M, M), jnp.float32))

    n_programs = B // NB
    grid_spec = pltpu.PrefetchScalarGridSpec(
        num_scalar_prefetch=1,
        grid=(n_programs,),
        in_specs=[
            pl.BlockSpec((NB, EMB), lambda i, u: (i, 0)),
            pl.BlockSpec((NB, S, D_IN), lambda i, u: (i, 0, 0)),
            pl.BlockSpec((EMB, D_PROC), lambda i, u: (0, 0)),
            pl.BlockSpec((1, D_PROC), lambda i, u: (0, 0)),
            pl.BlockSpec((E, RW, R * M), lambda i, u: (0, 0, 0)),
            pl.BlockSpec((E, R * M, RW), lambda i, u: (0, 0, 0)),
            pl.BlockSpec((R * M, RW), lambda i, u: (0, 0)),
            pl.BlockSpec((R * M, R * M), lambda i, u: (0, 0)),
            pl.BlockSpec((E, D_IN, RW), lambda i, u: (0, 0, 0)),
            pl.BlockSpec((E, D_PROC, RW), lambda i, u: (0, 0, 0)),
            pl.BlockSpec((E, RW), lambda i, u: (0, 0)),
        ],
        out_specs=[
            pl.BlockSpec((NB, S, R, WORD), lambda i, u: (i, 0, 0, 0)),
            pl.BlockSpec((S, R * M), lambda i, u: (0, 0)),
        ],
        scratch_shapes=[
            pltpu.VMEM((NB, S, R * M), jnp.bfloat16),
            pltpu.VMEM((E, D_IN, R * M), jnp.bfloat16),
            pltpu.VMEM((E, EMB, R * M), jnp.bfloat16),
            pltpu.VMEM((E, 1, R * M), jnp.float32),
        ],
    )
    body = functools.partial(_tc_body, S=S, R=R, WORD=WORD, E=E, RW=RW,
                             n_programs=n_programs)
    read_flat, fin = pl.pallas_call(
        body,
        grid_spec=grid_spec,
        out_shape=[
            jax.ShapeDtypeStruct((B, S, R, WORD), jnp.float32),
            jax.ShapeDtypeStruct((S, R * M), jnp.float32),
        ],
    )(uid, ue, inputs, W_proc, bp2, bd, bd2, o_mat, o2_mat, wt, wb, b_read)

    final_state = fin.reshape(S, R, M)
    return (read_flat, final_state)


# speedup vs baseline: 1.4245x; 1.2980x over previous
import functools

import numpy as np
import jax
import jax.numpy as jnp
from jax import lax
from jax.experimental import pallas as pl
from jax.experimental.pallas import tpu as pltpu
from jax.experimental.pallas import tpu_sc as plsc

NB = 32


def _sc_gather_build(V, D, B):
    info = plsc.get_sparse_core_info()
    NC, NS = info.num_cores, info.num_subcores
    NW = NC * NS
    assert D % info.num_lanes == 0 and B % (8 * NW) == 0
    b_per_w = B // NW
    mesh = plsc.VectorSubcoreMesh(core_axis_name="c", subcore_axis_name="s")

    @functools.partial(
        pl.kernel,
        mesh=mesh,
        compiler_params=pltpu.CompilerParams(use_tc_tiling_on_sc=False),
        out_type=jax.ShapeDtypeStruct((B, D), jnp.float32),
        scratch_types=[
            pltpu.VMEM((b_per_w,), jnp.int32),
            pltpu.VMEM((b_per_w, D), jnp.float32),
            pltpu.SemaphoreType.DMA,
        ],
    )
    def gather(table_hbm, idx_hbm, out_hbm, idx_v, rows_v, sem):
        wid = lax.axis_index("s") * NC + lax.axis_index("c")
        base = wid * b_per_w
        pltpu.sync_copy(idx_hbm.at[pl.ds(base, b_per_w)], idx_v)
        pltpu.async_copy(table_hbm.at[idx_v], rows_v, sem).wait()
        pltpu.sync_copy(rows_v, out_hbm.at[pl.ds(base, b_per_w)])

    return gather


def _tc_body(uid_ref, ue_ref, x_ref, wp_ref, bp_ref, bd_ref, bd2_ref, o_ref,
             o2_ref, wt_ref, wb_ref, br_ref, out_ref, fin_ref, e_scr,
             wtbd_scr, wpbbd_scr, bpbrbd_scr, *, S, R, WORD, E, RW,
             n_programs):
    i = pl.program_id(0)
    @pl.when(i == 0)
    def _():
        for s in range(E):
            t = jnp.dot(wb_ref[s], bd_ref[s],
                        preferred_element_type=jnp.float32)
            wtbd_scr[s] = jnp.dot(wt_ref[s], bd_ref[s],
                                  preferred_element_type=jnp.float32
                                  ).astype(jnp.bfloat16)
            wpbbd_scr[s] = jnp.dot(wp_ref[...], t,
                                   preferred_element_type=jnp.float32
                                   ).astype(jnp.bfloat16)
            bpbrbd_scr[s] = (
                jnp.dot(bp_ref[...], t, preferred_element_type=jnp.float32)
                + jnp.dot(br_ref[pl.ds(s, 1), :], bd_ref[s],
                          preferred_element_type=jnp.float32))
    uids = [uid_ref[i * NB + j] for j in range(NB)]
    sids = [lax.rem(u, E) for u in uids]
    for j in range(NB):
        sid = sids[j]
        ue_j = ue_ref[pl.ds(j, 1), :].astype(jnp.bfloat16)
        x = x_ref[j]
        sc = (jnp.dot(x, wtbd_scr[sid], preferred_element_type=jnp.float32)
              + jnp.dot(ue_j, wpbbd_scr[sid],
                        preferred_element_type=jnp.float32)
              + bpbrbd_scr[sid])
        e = jnp.exp(sc)
        e_scr[j] = e.astype(jnp.bfloat16)
        if j == NB - 1:
            @pl.when(i == n_programs - 1)
            def _():
                d512 = jnp.dot(e, o2_ref[...],
                               preferred_element_type=jnp.float32)
                fin_ref[...] = e / d512
    for j in range(NB):
        eb = e_scr[j]
        un = jnp.dot(eb, bd2_ref[sids[j]],
                     preferred_element_type=jnp.float32)
        den = jnp.dot(eb, o_ref[...], preferred_element_type=jnp.float32)
        out_ref[j] = un / den


def kernel(inputs, user_id, user_emb_table, W_proc, b_proc, shard_memory,
           W_read, b_read):
    B, S, D_IN = inputs.shape
    V, EMB = user_emb_table.shape
    E, M, WORD = shard_memory.shape
    RW = W_read.shape[-1]
    R = RW // WORD
    D_PROC = W_proc.shape[-1]

    uid = user_id.astype(jnp.int32)

    ue = _sc_gather_build(V, EMB, B)(user_emb_table, uid)

    wt = W_read[:, :D_IN, :]
    wb = W_read[:, D_IN:, :]
    bp2 = b_proc.reshape(1, D_PROC)

    memT = jnp.swapaxes(shard_memory, 1, 2) * (1.0 / np.sqrt(WORD))
    bd = jnp.zeros((E, R, WORD, R, M), jnp.float32)
    bd2 = jnp.zeros((E, R, M, R, WORD), jnp.float32)
    for r in range(R):
        bd = bd.at[:, r, :, r, :].set(memT)
        bd2 = bd2.at[:, r, :, r, :].set(shard_memory)
    bd = bd.reshape(E, RW, R * M)
    bd2 = bd2.reshape(E, R * M, RW).astype(jnp.bfloat16)
    o_mat = jnp.kron(jnp.eye(R, dtype=jnp.bfloat16),
                     jnp.ones((M, WORD), jnp.bfloat16))
    o2_mat = jnp.kron(jnp.eye(R, dtype=jnp.float32),
                      jnp.ones((M, M), jnp.float32))

    n_programs = B // NB
    grid_spec = pltpu.PrefetchScalarGridSpec(
        num_scalar_prefetch=1,
        grid=(n_programs,),
        in_specs=[
            pl.BlockSpec((NB, EMB), lambda i, u: (i, 0)),
            pl.BlockSpec((NB, S, D_IN), lambda i, u: (i, 0, 0)),
            pl.BlockSpec((EMB, D_PROC), lambda i, u: (0, 0)),
            pl.BlockSpec((1, D_PROC), lambda i, u: (0, 0)),
            pl.BlockSpec((E, RW, R * M), lambda i, u: (0, 0, 0)),
            pl.BlockSpec((E, R * M, RW), lambda i, u: (0, 0, 0)),
            pl.BlockSpec((R * M, RW), lambda i, u: (0, 0)),
            pl.BlockSpec((R * M, R * M), lambda i, u: (0, 0)),
            pl.BlockSpec((E, D_IN, RW), lambda i, u: (0, 0, 0)),
            pl.BlockSpec((E, D_PROC, RW), lambda i, u: (0, 0, 0)),
            pl.BlockSpec((E, RW), lambda i, u: (0, 0)),
        ],
        out_specs=[
            pl.BlockSpec((NB, S, RW), lambda i, u: (i, 0, 0)),
            pl.BlockSpec((S, R * M), lambda i, u: (0, 0)),
        ],
        scratch_shapes=[
            pltpu.VMEM((NB, S, R * M), jnp.bfloat16),
            pltpu.VMEM((E, D_IN, R * M), jnp.bfloat16),
            pltpu.VMEM((E, EMB, R * M), jnp.bfloat16),
            pltpu.VMEM((E, 1, R * M), jnp.float32),
        ],
    )
    body = functools.partial(_tc_body, S=S, R=R, WORD=WORD, E=E, RW=RW,
                             n_programs=n_programs)
    read_flat, fin = pl.pallas_call(
        body,
        grid_spec=grid_spec,
        out_shape=[
            jax.ShapeDtypeStruct((B, S, RW), jnp.float32),
            jax.ShapeDtypeStruct((S, R * M), jnp.float32),
        ],
    )(uid, ue, inputs.astype(jnp.bfloat16), W_proc, bp2, bd, bd2, o_mat,
      o2_mat, wt, wb, b_read)

    read_words = read_flat.reshape(B, S, R, WORD)
    final_state = fin.reshape(S, R, M)
    return (read_words, final_state)
